# baseline (speedup 1.0000x reference)
import functools

import jax
import jax.numpy as jnp
from jax import lax
from jax.experimental import pallas as pl
from jax.experimental.pallas import tpu as pltpu
from jax.experimental.pallas import tpu_sc as plsc

D = 64
NC, NS = 2, 16
NW = NC * NS
BT = 128
LANES = 16
NG = BT // LANES

_MESH = plsc.VectorSubcoreMesh(
    core_axis_name="c", subcore_axis_name="s", num_cores=NC, num_subcores=NS
)


def _make_emb(seq, batch):
    n_btiles = batch // BT
    assert n_btiles == NW

    @functools.partial(
        pl.kernel,
        out_type=jax.ShapeDtypeStruct((seq, D // 8, n_btiles, 8, 128), jnp.float32),
        mesh=_MESH,
        scratch_types=[
            pltpu.VMEM((seq, BT), jnp.int32),
            pltpu.VMEM((D, seq), jnp.float32),
            pltpu.VMEM((2, BT, D), jnp.float32),
            pltpu.VMEM((2, D // 8, 8, 128), jnp.float32),
            pltpu.SemaphoreType.DMA,
            pltpu.SemaphoreType.DMA,
            pltpu.SemaphoreType.DMA,
            pltpu.SemaphoreType.DMA,
        ],
        compiler_params=pltpu.CompilerParams(
            use_tc_tiling_on_sc=False, needs_layout_passes=False
        ),
    )
    def emb(xt_hbm, tok_hbm, post_hbm, out_hbm, xv, posv, gv, ov, g0, g1, w0, w1):
        gs, ws = [g0, g1], [w0, w1]
        wid = lax.axis_index("s") * NC + lax.axis_index("c")
        b0 = wid * BT

        pltpu.sync_copy(xt_hbm.at[:, pl.ds(b0, BT)], xv)
        pltpu.sync_copy(post_hbm, posv)

        iota = lax.iota(jnp.int32, LANES)
        jrows = [iota + g * LANES for g in range(NG)]

        def fire_gather(i, b):
            pltpu.async_copy(tok_hbm.at[xv.at[i]], gv.at[b], gs[b])

        def wait_gather(b):
            pltpu.make_async_copy(tok_hbm.at[pl.ds(0, BT)], gv.at[b], gs[b]).wait()

        def wait_wb(b):
            pltpu.make_async_copy(ov.at[b], out_hbm.at[0, :, 0], ws[b]).wait()

        fire_gather(0, 0)

        @pl.loop(0, seq, step=2)
        def _(s2):
            for b in range(2):
                nb = 1 - b
                i = s2 + b

                @pl.when(i + 1 < seq)
                def _():
                    fire_gather(i + 1, nb)

                wait_gather(b)

                @pl.when(i >= 2)
                def _():
                    wait_wb(b)

                s_vec = jnp.broadcast_to(i, (LANES,)).astype(jnp.int32)

                for d in range(D):
                    d_vec = jnp.full((LANES,), d, jnp.int32)
                    ps = plsc.load_gather(posv, [d_vec, s_vec])
                    for g in range(NG):
                        v = plsc.load_gather(gv.at[b], [jrows[g], d_vec]) + ps
                        ov[b, d // 8, d % 8, pl.ds(g * LANES, LANES)] = v

                pltpu.async_copy(ov.at[b], out_hbm.at[i, :, wid], ws[b])

        wait_wb(0)
        wait_wb(1)

    return emb


def kernel(x, tok_emb, pos_emb):
    batch, seq = x.shape
    xt = jnp.transpose(x)
    post = jnp.transpose(pos_emb)
    out5 = _make_emb(seq, batch)(xt, tok_emb.astype(jnp.float32), post)
    out = out5.transpose((2, 4, 0, 1, 3)).reshape(batch, seq, D)
    return out


# speedup vs baseline: 2.7897x; 2.7897x over previous
import functools

import jax
import jax.numpy as jnp
from jax import lax
from jax.experimental import pallas as pl
from jax.experimental.pallas import tpu as pltpu
from jax.experimental.pallas import tpu_sc as plsc

D = 64
NC, NS = 2, 16
NW = NC * NS
BT = 128
LANES = 16
NG = BT // LANES

_MESH = plsc.VectorSubcoreMesh(
    core_axis_name="c", subcore_axis_name="s", num_cores=NC, num_subcores=NS
)


def _make_emb(seq, batch):
    n_btiles = batch // BT
    assert n_btiles == NW

    PITCH = 129

    @functools.partial(
        pl.kernel,
        out_type=jax.ShapeDtypeStruct((seq, D // 8, n_btiles, 8, 128), jnp.float32),
        mesh=_MESH,
        scratch_types=[
            pltpu.VMEM((seq, BT), jnp.int32),
            pltpu.VMEM((seq, D), jnp.float32),
            pltpu.VMEM((2, BT, D), jnp.float32),
            pltpu.VMEM((2, D // 8, 8, PITCH), jnp.float32),
            pltpu.SemaphoreType.DMA,
            pltpu.SemaphoreType.DMA,
            pltpu.SemaphoreType.DMA,
            pltpu.SemaphoreType.DMA,
        ],
        compiler_params=pltpu.CompilerParams(
            use_tc_tiling_on_sc=False, needs_layout_passes=False
        ),
    )
    def emb(xt_hbm, tok_hbm, pos_hbm, out_hbm, xv, posv, gv, ov, g0, g1, w0, w1):
        gs, ws = [g0, g1], [w0, w1]
        wid = lax.axis_index("s") * NC + lax.axis_index("c")
        b0 = wid * BT

        pltpu.sync_copy(xt_hbm.at[:, pl.ds(b0, BT)], xv)
        pltpu.sync_copy(pos_hbm, posv)

        iota = lax.iota(jnp.int32, LANES)
        dt_k = [iota // 8 + 2 * k for k in range(D // LANES)]
        dr_k = iota % 8

        def fire_gather(i, b):
            pltpu.async_copy(tok_hbm.at[xv.at[i]], gv.at[b], gs[b])

        def wait_gather(b):
            pltpu.make_async_copy(tok_hbm.at[pl.ds(0, BT)], gv.at[b], gs[b]).wait()

        def wait_wb(b):
            pltpu.make_async_copy(
                ov.at[b].at[:, :, pl.ds(0, 128)], out_hbm.at[0, :, 0], ws[b]
            ).wait()

        fire_gather(0, 0)

        @pl.loop(0, seq, step=2)
        def _(s2):
            for b in range(2):
                nb = 1 - b
                i = s2 + b

                @pl.when(i + 1 < seq)
                def _():
                    fire_gather(i + 1, nb)

                wait_gather(b)

                @pl.when(i >= 2)
                def _():
                    wait_wb(b)

                pos_k = [posv[i, pl.ds(k * LANES, LANES)] for k in range(D // LANES)]
                ovb = ov.at[b]
                gvb = gv.at[b]

                @functools.partial(plsc.parallel_loop, 0, BT, unroll=2)
                def _(j):
                    jv = jnp.broadcast_to(j, (LANES,)).astype(jnp.int32)
                    for k in range(D // LANES):
                        v = gvb[j, pl.ds(k * LANES, LANES)] + pos_k[k]
                        plsc.store_scatter(ovb, [dt_k[k], dr_k, jv], v)

                pltpu.async_copy(
                    ov.at[b].at[:, :, pl.ds(0, 128)], out_hbm.at[i, :, wid], ws[b]
                )

        wait_wb(0)
        wait_wb(1)

    return emb


def kernel(x, tok_emb, pos_emb):
    batch, seq = x.shape
    xt = jnp.transpose(x)
    out5 = _make_emb(seq, batch)(xt, tok_emb.astype(jnp.float32), pos_emb)
    out = out5.transpose((2, 4, 0, 1, 3)).reshape(batch, seq, D)
    return out
